# baseline (speedup 1.0000x reference)
import jax
import jax.numpy as jnp
from jax import lax
from jax.experimental import pallas as pl
from jax.experimental.pallas import tpu as pltpu
from jax.experimental.pallas import tpu_sc as plsc

VOCAB = 1000000
MAXLEN = 200
D = 64
BATCH = 4096
SEQ = 200

NC = 2
NS = 16
NW = NC * NS

N = BATCH * SEQ
SEQ_PER_W = BATCH // NW
ROWS_PER_W = SEQ_PER_W * SEQ
NBUF = 4


def _sc_body(idx_hbm, tok_hbm, pos_hbm, out_hbm, pos_sh, idx_v, rows_bf,
             rows_f, g0, g1, g2, g3, w0, w1, w2, w3, p0, p1, p2, p3):
    gsem = (g0, g1, g2, g3)
    wsem = (w0, w1, w2, w3)
    psem = (p0, p1, p2, p3)
    cid = lax.axis_index("c")
    sid = lax.axis_index("s")
    wid = sid * NC + cid
    base0 = wid * ROWS_PER_W

    @pl.when(sid == 0)
    def _():
        pltpu.sync_copy(pos_hbm, pos_sh)

    pltpu.sync_copy(idx_hbm.at[pl.ds(base0, ROWS_PER_W)], idx_v)
    plsc.subcore_barrier()

    def prefill(b):
        pass

    def issue_gather(g, b):
        pltpu.async_copy(tok_hbm.at[idx_v.at[pl.ds(g * SEQ, SEQ)]],
                         rows_bf.at[b], gsem[b])

    def wait_gather(b):
        pltpu.make_async_copy(tok_hbm.at[idx_v.at[pl.ds(0, SEQ)]],
                              rows_bf.at[b], gsem[b]).wait()

    def issue_write(g, b):
        pltpu.async_copy(rows_f.at[b], out_hbm.at[pl.ds(base0 + g * SEQ, SEQ)],
                         wsem[b])

    def wait_write(b):
        pltpu.make_async_copy(rows_f.at[b], out_hbm.at[pl.ds(0, SEQ)],
                              wsem[b]).wait()

    def unpack_chunk(b):
        src = rows_bf.at[b]
        dst = rows_f.at[b]

        def body(r, c):
            for h in range(D // 32):
                packed = src[r, pl.ds(32 * h, 32)]
                lo, hi = plsc.unpack(packed, format=plsc.PackFormat.INTERLEAVED,
                                     preferred_element_type=jnp.float32)
                dst[r, pl.ds(32 * h, 16)] = lo
                dst[r, pl.ds(32 * h + 16, 16)] = hi
            return c

        if False:
            lax.fori_loop(0, SEQ, body, 0, unroll=2)

    def step(g, b, retire=True):
        wait_gather(b)
        if retire:
            wait_write(b)
        unpack_chunk(b)
        issue_write(g, b)

    def stage(g, b):
        prefill((b + 3) % NBUF)
        issue_gather(g + 2, (b + 2) % NBUF)

    for b in range(3):
        prefill(b)
    issue_gather(0, 0)
    issue_gather(1, 1)

    for g in range(NBUF):
        step(g, g % NBUF, retire=False)
        stage(g, g % NBUF)

    def qbody(q, c):
        for b in range(NBUF):
            g = q * NBUF + b
            step(g, b)
            stage(g, b)
        return c

    lax.fori_loop(1, SEQ_PER_W // NBUF - 1, qbody, 0)

    for g in range(SEQ_PER_W - NBUF, SEQ_PER_W):
        step(g, g % NBUF)
        if g + 3 < SEQ_PER_W:
            prefill((g + 3) % NBUF)
        if g + 2 < SEQ_PER_W:
            issue_gather(g + 2, (g + 2) % NBUF)

    for b in range(NBUF):
        wait_write(b)


def _shuffle_bf16(table):
    return table.astype(jnp.bfloat16)


@jax.jit
def _run(idx_flat, token_table, pos_table):
    tok_shuf = _shuffle_bf16(token_table)
    pos_shuf = _shuffle_bf16(pos_table)
    mesh = plsc.VectorSubcoreMesh(core_axis_name="c", subcore_axis_name="s")
    f = pl.kernel(
        _sc_body,
        out_type=jax.ShapeDtypeStruct((N, D), jnp.float32),
        mesh=mesh,
        scratch_types=[
            pltpu.VMEM_SHARED((MAXLEN, D), jnp.bfloat16),
            pltpu.VMEM((ROWS_PER_W,), jnp.int32),
            pltpu.VMEM((NBUF, SEQ, D), jnp.bfloat16),
            pltpu.VMEM((NBUF, SEQ, D), jnp.float32),
        ] + [pltpu.SemaphoreType.DMA] * (3 * NBUF),
        compiler_params=pltpu.CompilerParams(use_tc_tiling_on_sc=False, needs_layout_passes=False),
    )
    return f(idx_flat, tok_shuf, pos_shuf)


def kernel(inputs, token_table, pos_table):
    idx_flat = inputs.astype(jnp.int32).reshape(N)
    out = _run(idx_flat, token_table, pos_table)
    return out.reshape(BATCH, SEQ, D)


# speedup vs baseline: 1.1548x; 1.1548x over previous
import jax
import jax.numpy as jnp
from jax import lax
from jax.experimental import pallas as pl
from jax.experimental.pallas import tpu as pltpu
from jax.experimental.pallas import tpu_sc as plsc

VOCAB = 1000000
MAXLEN = 200
D = 64
BATCH = 4096
SEQ = 200

NC = 2
NS = 16
NW = NC * NS

N = BATCH * SEQ
SEQ_PER_W = BATCH // NW
ROWS_PER_W = SEQ_PER_W * SEQ
NBUF = 6
IDX_HEAD = 8 * SEQ


def _sc_body(idx_hbm, tok_hbm, pos_hbm, out_hbm, pos_sh, idx_v, rows_v,
             isem, *sems):
    gsem = sems[:NBUF]
    wsem = sems[NBUF:2 * NBUF]
    psem = sems[2 * NBUF:]
    cid = lax.axis_index("c")
    sid = lax.axis_index("s")
    wid = sid * NC + cid
    base0 = wid * ROWS_PER_W

    @pl.when(sid == 0)
    def _():
        pltpu.sync_copy(pos_hbm, pos_sh)

    pltpu.sync_copy(idx_hbm.at[pl.ds(base0, IDX_HEAD)],
                    idx_v.at[pl.ds(0, IDX_HEAD)])
    pltpu.async_copy(idx_hbm.at[pl.ds(base0 + IDX_HEAD, ROWS_PER_W - IDX_HEAD)],
                     idx_v.at[pl.ds(IDX_HEAD, ROWS_PER_W - IDX_HEAD)], isem)
    plsc.subcore_barrier()

    def prefill(b):
        pltpu.async_copy(pos_sh, rows_v.at[b], psem[b])

    def issue_gather(g, b):
        pltpu.make_async_copy(pos_sh, rows_v.at[b], psem[b]).wait()
        pltpu.async_copy(tok_hbm.at[idx_v.at[pl.ds(g * SEQ, SEQ)]],
                         rows_v.at[b], gsem[b], add=True)

    def wait_gather(b):
        pltpu.make_async_copy(tok_hbm.at[idx_v.at[pl.ds(0, SEQ)]],
                              rows_v.at[b], gsem[b]).wait()

    def issue_write(g, b):
        pltpu.async_copy(rows_v.at[b], out_hbm.at[pl.ds(base0 + g * SEQ, SEQ)],
                         wsem[b])

    def wait_write(b):
        pltpu.make_async_copy(rows_v.at[b], out_hbm.at[pl.ds(0, SEQ)],
                              wsem[b]).wait()

    def step(g, b):
        wait_gather(b)
        issue_write(g, b)

    for b in range(NBUF - 1):
        prefill(b)
    for g in range(NBUF - 2):
        issue_gather(g, g)

    for g in range(NBUF):
        step(g, g)
        if g == 3:
            pltpu.make_async_copy(
                idx_hbm.at[pl.ds(0, ROWS_PER_W - IDX_HEAD)],
                idx_v.at[pl.ds(IDX_HEAD, ROWS_PER_W - IDX_HEAD)], isem).wait()
        bp = (g + 5) % NBUF
        if g >= 1:
            wait_write(bp)
        prefill(bp)
        issue_gather(g + 4, (g + 4) % NBUF)

    def qbody(q, c):
        for b in range(NBUF):
            g = q * NBUF + b
            step(g, b)
            bp = (b + 5) % NBUF
            wait_write(bp)
            prefill(bp)
            issue_gather(g + 4, (b + 4) % NBUF)
        return c

    nq = (SEQ_PER_W - 4) // NBUF
    lax.fori_loop(1, nq, qbody, 0)

    for g in range(nq * NBUF, SEQ_PER_W):
        step(g, g % NBUF)
        if g + 5 < SEQ_PER_W:
            bp = (g + 5) % NBUF
            wait_write(bp)
            prefill(bp)
        if g + 4 < SEQ_PER_W:
            issue_gather(g + 4, (g + 4) % NBUF)

    for b in range(NBUF):
        wait_write(b)


@jax.jit
def _run(idx_flat, token_table, pos_table):
    mesh = plsc.VectorSubcoreMesh(core_axis_name="c", subcore_axis_name="s")
    f = pl.kernel(
        _sc_body,
        out_type=jax.ShapeDtypeStruct((N, D), jnp.float32),
        mesh=mesh,
        scratch_types=[
            pltpu.VMEM_SHARED((MAXLEN, D), jnp.float32),
            pltpu.VMEM((ROWS_PER_W,), jnp.int32),
            pltpu.VMEM((NBUF, SEQ, D), jnp.float32),
        ] + [pltpu.SemaphoreType.DMA] * (1 + 3 * NBUF),
        compiler_params=pltpu.CompilerParams(use_tc_tiling_on_sc=False),
    )
    return f(idx_flat, token_table, pos_table)


def kernel(inputs, token_table, pos_table):
    idx_flat = inputs.astype(jnp.int32).reshape(N)
    out = _run(idx_flat, token_table, pos_table)
    return out.reshape(BATCH, SEQ, D)
